# baseline (speedup 1.0000x reference)
import jax
import jax.numpy as jnp
from jax import lax
from jax.experimental import pallas as pl
from jax.experimental.pallas import tpu as pltpu
from jax.experimental.pallas import tpu_sc as plsc

_NC = 2
_NS = 16
_NW = _NC * _NS

_BW = 128


def _gather_body(idx_hbm, table_hbm, out_hbm, idx_v, buf_a, buf_b,
                 gsem, oa_sem, ob_sem):
    seq = idx_hbm.shape[0]
    n_pairs = seq // 2

    wid = lax.axis_index("s") * _NC + lax.axis_index("c")
    col0 = pl.multiple_of(wid * _BW, _BW)

    pltpu.sync_copy(idx_hbm.at[:, pl.ds(col0, _BW)], idx_v)

    def run_step(s, buf, osem):
        pltpu.async_copy(table_hbm.at[idx_v.at[s]], buf, gsem).wait()
        pltpu.async_copy(buf, out_hbm.at[s, pl.ds(col0, _BW)], osem)

    def pair_body(i, carry):
        sa = 2 * i
        sb = 2 * i + 1

        @pl.when(i > 0)
        def _():
            pltpu.make_async_copy(buf_a, out_hbm.at[sa, pl.ds(col0, _BW)],
                                  oa_sem).wait()

        run_step(sa, buf_a, oa_sem)

        @pl.when(i > 0)
        def _():
            pltpu.make_async_copy(buf_b, out_hbm.at[sb, pl.ds(col0, _BW)],
                                  ob_sem).wait()

        run_step(sb, buf_b, ob_sem)
        return carry

    lax.fori_loop(0, n_pairs, pair_body, 0)

    last = seq - 1
    pltpu.make_async_copy(buf_a, out_hbm.at[last, pl.ds(col0, _BW)],
                          oa_sem).wait()
    pltpu.make_async_copy(buf_b, out_hbm.at[last, pl.ds(col0, _BW)],
                          ob_sem).wait()


@jax.jit
def _gather(idx_t, table):
    seq, batch = idx_t.shape
    d = table.shape[1]
    mesh = plsc.VectorSubcoreMesh(core_axis_name="c", subcore_axis_name="s")
    out = pl.kernel(
        _gather_body,
        out_type=jax.ShapeDtypeStruct((seq, batch, d), jnp.float32),
        mesh=mesh,
        scratch_types=[
            pltpu.VMEM((seq, _BW), jnp.int32),
            pltpu.VMEM((_BW, d), jnp.float32),
            pltpu.VMEM((_BW, d), jnp.float32),
            pltpu.SemaphoreType.DMA,
            pltpu.SemaphoreType.DMA,
            pltpu.SemaphoreType.DMA,
        ],
        compiler_params=pltpu.CompilerParams(use_tc_tiling_on_sc=False),
    )(idx_t, table)
    return out.transpose(1, 0, 2)


def kernel(indices, table):
    return _gather(indices.T, table)
